# baseline (speedup 1.0000x reference)
import functools

import jax
import jax.numpy as jnp
from jax import lax
from jax.experimental import pallas as pl
from jax.experimental.pallas import tpu as pltpu
from jax.experimental.pallas import tpu_sc as plsc

EMB = 64
NROWS = 1000000
B_USERS = 16384
N_NEG = 65536
NEG_PER_USER = N_NEG // B_USERS
NUM_WORKERS = 32
USERS_PER_WORKER = B_USERS // NUM_WORKERS
SUB = 128
NSUB = USERS_PER_WORKER // SUB
IDX_LANES = 128
ROWW = 2 * EMB
NBLK = (NROWS + 127) // 128
BLK_PER_SUBCORE = (NBLK + 15) // 16


def _relayout_table(tabT, out, Vblk, Oblk, s, semi, semo):
    iota = lax.broadcasted_iota(jnp.int32, (16,), 0)
    kvecs = [i * 16 + iota for i in range(EMB // 16)]
    lo = s * BLK_PER_SUBCORE
    hi = jnp.minimum(NBLK, lo + BLK_PER_SUBCORE)

    def colof(j):
        return pl.multiple_of(j * 128, 128)

    def in_cp(j, slot):
        return pltpu.make_async_copy(
            tabT.at[:, pl.ds(colof(j), 128)], Vblk.at[slot], semi.at[slot])

    def out_cp(j, slot):
        return pltpu.make_async_copy(
            Oblk.at[slot],
            out.at[pl.ds(pl.multiple_of(colof(j) // 2, 64), 64)],
            semo.at[slot])

    def step(j, slot, first):
        in_cp(j, slot).wait()
        if not first:
            out_cp(j, slot).wait()

        Vs, Os = Vblk.at[slot], Oblk.at[slot]

        def rbody(r, _):
            for h in range(2):
                e = jnp.full((16,), 2 * r + h, jnp.int32)
                for i in range(EMB // 16):
                    v = plsc.load_gather(Vs, [kvecs[i], e])
                    Os[r, pl.ds(h * EMB + i * 16, 16)] = v
            return 0

        lax.fori_loop(0, 64, rbody, 0)
        out_cp(j, slot).start()

        @pl.when(j + 2 < hi)
        def _():
            in_cp(j + 2, slot).start()

    in_cp(lo, 0).start()
    in_cp(lo + 1, 1).start()
    step(lo, 0, True)
    step(lo + 1, 1, True)

    def jbody(jj, _):
        j = lo + 2 * jj

        @pl.when(j < hi)
        def _():
            step(j, 0, False)

        @pl.when(j + 1 < hi)
        def _():
            step(j + 1, 1, False)

        return 0

    lax.fori_loop(1, (BLK_PER_SUBCORE + 1) // 2, jbody, 0)
    out_cp(hi - 2, 0).wait()
    out_cp(hi - 1, 1).wait()


def _k1_body(uT_hbm, iT_hbm, u_out, i_out, Vblk, Oblk, semi, semo):
    c = lax.axis_index("c")
    s = lax.axis_index("s")

    @pl.when(c == 0)
    def _():
        _relayout_table(uT_hbm, u_out, Vblk, Oblk, s, semi, semo)

    @pl.when(c == 1)
    def _():
        _relayout_table(iT_hbm, i_out, Vblk, Oblk, s, semi, semo)


_k1 = functools.partial(
    pl.kernel,
    mesh=plsc.VectorSubcoreMesh(core_axis_name="c", subcore_axis_name="s"),
    compiler_params=pltpu.CompilerParams(needs_layout_passes=False),
    out_type=(
        jax.ShapeDtypeStruct((NBLK * 64, ROWW), jnp.float32),
        jax.ShapeDtypeStruct((NBLK * 64, ROWW), jnp.float32),
    ),
    scratch_types=[
        pltpu.VMEM((2, EMB, 128), jnp.float32),
        pltpu.VMEM((2, 64, ROWW), jnp.float32),
        pltpu.SemaphoreType.DMA((2,)),
        pltpu.SemaphoreType.DMA((2,)),
    ],
)(_k1_body)


def _k2_body(ue_hbm, pe_hbm, ne_hbm, ur_hbm, pr_hbm, nr_hbm,
             uemb_hbm, iemb_hbm, pos_out, neg_out,
             ue_v, pe_v, ne_v, ur_v, pr_v, nr_v, U, P, N, pos_v, neg_v, sem):
    w = lax.axis_index("s") * 2 + lax.axis_index("c")
    iota = lax.broadcasted_iota(jnp.int32, (16,), 0)

    pltpu.sync_copy(ue_hbm.at[pl.ds(w * NSUB, NSUB)], ue_v)
    pltpu.sync_copy(pe_hbm.at[pl.ds(w * NSUB, NSUB)], pe_v)
    pltpu.sync_copy(ne_hbm.at[pl.ds(w * (NSUB * 4), NSUB * 4)], ne_v)
    pltpu.sync_copy(ur_hbm.at[pl.ds(w * NSUB, NSUB)], ur_v)
    pltpu.sync_copy(pr_hbm.at[pl.ds(w * NSUB, NSUB)], pr_v)
    pltpu.sync_copy(nr_hbm.at[pl.ds(w * (NSUB * 4), NSUB * 4)], nr_v)

    for c in range(NSUB):
        cps = [
            pltpu.async_copy(uemb_hbm.at[ur_v.at[c]], U, sem),
            pltpu.async_copy(iemb_hbm.at[pr_v.at[c]], P, sem),
        ]
        for j in range(4):
            cps.append(pltpu.async_copy(
                iemb_hbm.at[nr_v.at[4 * c + j]],
                N.at[pl.ds(IDX_LANES * j, IDX_LANES)], sem))
        for cp in cps:
            cp.wait()

        def gbody(g, _):
            urow = iota + g * 16
            nrows = [4 * iota + g * 64 + n for n in range(NEG_PER_USER)]
            ub = (ue_v[c, pl.ds(g * 16, 16)] & 1) << 6
            pb = (pe_v[c, pl.ds(g * 16, 16)] & 1) << 6
            nbs = [(plsc.load_gather(ne_v, [jnp.full((16,), 4 * c, jnp.int32)
                                            + (64 * g + 4 * iota + n) // 128,
                                            (64 * g + 4 * iota + n) % 128])
                    & 1) << 6
                   for n in range(NEG_PER_USER)]
            u0 = plsc.load_gather(U, [urow, ub])
            acc_p = u0 + plsc.load_gather(P, [urow, pb + 63])
            accs = [u0 + plsc.load_gather(N, [nr, nb + 63])
                    for nr, nb in zip(nrows, nbs)]
            for k in range(EMB - 1):
                ucol = plsc.load_gather(U, [urow, ub + (k + 1)])
                acc_p = acc_p + ucol * plsc.load_gather(P, [urow, pb + k])
                for n in range(NEG_PER_USER):
                    accs[n] = accs[n] + ucol * plsc.load_gather(
                        N, [nrows[n], nbs[n] + k])
            pos_v[pl.ds(g * 16, 16)] = 1.0 / (1.0 + jnp.exp(-acc_p))
            for n in range(NEG_PER_USER):
                plsc.store_scatter(neg_v, [4 * iota + g * 64 + n],
                                   1.0 / (1.0 + jnp.exp(-accs[n])))
            return 0

        lax.fori_loop(0, SUB // 16, gbody, 0)

        pltpu.sync_copy(pos_v,
                        pos_out.at[pl.ds(w * USERS_PER_WORKER + c * SUB, SUB)])
        pltpu.sync_copy(neg_v,
                        neg_out.at[pl.ds((w * NSUB + c) * (SUB * 4), SUB * 4)])


_k2 = functools.partial(
    pl.kernel,
    mesh=plsc.VectorSubcoreMesh(core_axis_name="c", subcore_axis_name="s"),
    compiler_params=pltpu.CompilerParams(needs_layout_passes=False),
    out_type=(
        jax.ShapeDtypeStruct((B_USERS,), jnp.float32),
        jax.ShapeDtypeStruct((N_NEG,), jnp.float32),
    ),
    scratch_types=[
        pltpu.VMEM((NSUB, IDX_LANES), jnp.int32),
        pltpu.VMEM((NSUB, IDX_LANES), jnp.int32),
        pltpu.VMEM((NSUB * 4, IDX_LANES), jnp.int32),
        pltpu.VMEM((NSUB, IDX_LANES), jnp.int32),
        pltpu.VMEM((NSUB, IDX_LANES), jnp.int32),
        pltpu.VMEM((NSUB * 4, IDX_LANES), jnp.int32),
        pltpu.VMEM((SUB, ROWW), jnp.float32),
        pltpu.VMEM((SUB, ROWW), jnp.float32),
        pltpu.VMEM((SUB * 4, ROWW), jnp.float32),
        pltpu.VMEM((SUB,), jnp.float32),
        pltpu.VMEM((SUB * 4,), jnp.float32),
        pltpu.SemaphoreType.DMA,
    ],
)(_k2_body)


@jax.jit
def kernel(batch_pos, neg_item_index, user_embedding, item_embedding):
    user_num = user_embedding.shape[0]
    uidx = batch_pos[:, 0].astype(jnp.int32)
    pidx = (batch_pos[:, 1] - user_num).astype(jnp.int32)
    nidx = (neg_item_index - user_num).astype(jnp.int32)
    u2, i2 = _k1(user_embedding.T, item_embedding.T)
    r = lambda a: a.reshape(-1, IDX_LANES)
    pos, neg = _k2(r(uidx), r(pidx), r(nidx),
                   r(uidx >> 1), r(pidx >> 1), r(nidx >> 1),
                   u2, i2)
    return pos.reshape(-1, 1), neg.reshape(-1, 1)


# speedup vs baseline: 2.7772x; 2.7772x over previous
import functools

import jax
import jax.numpy as jnp
from jax import lax
from jax.experimental import pallas as pl
from jax.experimental.pallas import tpu as pltpu
from jax.experimental.pallas import tpu_sc as plsc

EMB = 64
NROWS = 1000000
B_USERS = 16384
N_NEG = 65536
NEG_PER_USER = N_NEG // B_USERS
NUM_WORKERS = 32
USERS_PER_WORKER = B_USERS // NUM_WORKERS
SUB = 128
NSUB = USERS_PER_WORKER // SUB
IDX_LANES = 128
ROWW = 2 * EMB
NBLK = (NROWS + 127) // 128
BLK_PER_SUBCORE = (NBLK + 15) // 16


def _relayout_table(tabT, out, Vblk, Oblk, s, semi, semo):
    iota = lax.broadcasted_iota(jnp.int32, (16,), 0)
    kvecs = [i * 16 + iota for i in range(EMB // 16)]
    lo = s * BLK_PER_SUBCORE
    hi = jnp.minimum(NBLK, lo + BLK_PER_SUBCORE)

    def colof(j):
        return pl.multiple_of(j * 128, 128)

    def in_cp(j, slot):
        return pltpu.make_async_copy(
            tabT.at[:, pl.ds(colof(j), 128)], Vblk.at[slot], semi.at[slot])

    def out_cp(j, slot):
        return pltpu.make_async_copy(
            Oblk.at[slot],
            out.at[pl.ds(pl.multiple_of(colof(j) // 2, 64), 64)],
            semo.at[slot])

    def step(j, slot, first):
        in_cp(j, slot).wait()
        if not first:
            out_cp(j, slot).wait()

        Vs, Os = Vblk.at[slot], Oblk.at[slot]

        def pbody(t, _):
            e0 = t * 16
            for d in range(16):
                m = (iota + d) & 15
                ecol = e0 + m
                orow = (e0 >> 1) + (m >> 1)
                ocol = (m & 1) << 6
                for i in range(EMB // 16):
                    v = plsc.load_gather(Vs, [kvecs[i], ecol])
                    plsc.store_scatter(Os, [orow, ocol + kvecs[i]], v)
            return 0

        lax.fori_loop(0, 8, pbody, 0)
        out_cp(j, slot).start()

        @pl.when(j + 2 < hi)
        def _():
            in_cp(j + 2, slot).start()

    in_cp(lo, 0).start()
    in_cp(lo + 1, 1).start()
    step(lo, 0, True)
    step(lo + 1, 1, True)

    def jbody(jj, _):
        j = lo + 2 * jj

        @pl.when(j < hi)
        def _():
            step(j, 0, False)

        @pl.when(j + 1 < hi)
        def _():
            step(j + 1, 1, False)

        return 0

    lax.fori_loop(1, (BLK_PER_SUBCORE + 1) // 2, jbody, 0)
    out_cp(hi - 2, 0).wait()
    out_cp(hi - 1, 1).wait()


def _k1_body(uT_hbm, iT_hbm, u_out, i_out, Vblk, Oblk, semi, semo):
    c = lax.axis_index("c")
    s = lax.axis_index("s")

    @pl.when(c == 0)
    def _():
        _relayout_table(uT_hbm, u_out, Vblk, Oblk, s, semi, semo)

    @pl.when(c == 1)
    def _():
        _relayout_table(iT_hbm, i_out, Vblk, Oblk, s, semi, semo)


_k1 = functools.partial(
    pl.kernel,
    mesh=plsc.VectorSubcoreMesh(core_axis_name="c", subcore_axis_name="s"),
    compiler_params=pltpu.CompilerParams(needs_layout_passes=False),
    out_type=(
        jax.ShapeDtypeStruct((NBLK * 64, ROWW), jnp.float32),
        jax.ShapeDtypeStruct((NBLK * 64, ROWW), jnp.float32),
    ),
    scratch_types=[
        pltpu.VMEM((2, EMB, 128), jnp.float32),
        pltpu.VMEM((2, 64, ROWW), jnp.float32),
        pltpu.SemaphoreType.DMA((2,)),
        pltpu.SemaphoreType.DMA((2,)),
    ],
)(_k1_body)


def _k2_body(ue_hbm, pe_hbm, ne_hbm, ur_hbm, pr_hbm, nr_hbm,
             uemb_hbm, iemb_hbm, pos_out, neg_out,
             ue_v, pe_v, ne_v, ur_v, pr_v, nr_v, U, P, N, pos_v, neg_v, sem):
    w = lax.axis_index("s") * 2 + lax.axis_index("c")
    iota = lax.broadcasted_iota(jnp.int32, (16,), 0)

    pltpu.sync_copy(ue_hbm.at[pl.ds(w * NSUB, NSUB)], ue_v)
    pltpu.sync_copy(pe_hbm.at[pl.ds(w * NSUB, NSUB)], pe_v)
    pltpu.sync_copy(ne_hbm.at[pl.ds(w * (NSUB * 4), NSUB * 4)], ne_v)
    pltpu.sync_copy(ur_hbm.at[pl.ds(w * NSUB, NSUB)], ur_v)
    pltpu.sync_copy(pr_hbm.at[pl.ds(w * NSUB, NSUB)], pr_v)
    pltpu.sync_copy(nr_hbm.at[pl.ds(w * (NSUB * 4), NSUB * 4)], nr_v)

    for c in range(NSUB):
        cps = [
            pltpu.async_copy(uemb_hbm.at[ur_v.at[c]], U, sem),
            pltpu.async_copy(iemb_hbm.at[pr_v.at[c]], P, sem),
        ]
        for j in range(4):
            cps.append(pltpu.async_copy(
                iemb_hbm.at[nr_v.at[4 * c + j]],
                N.at[pl.ds(IDX_LANES * j, IDX_LANES)], sem))
        for cp in cps:
            cp.wait()

        def gbody(g, _):
            urow = iota + g * 16
            nrows = [4 * iota + g * 64 + n for n in range(NEG_PER_USER)]
            ub = (ue_v[c, pl.ds(g * 16, 16)] & 1) << 6
            pb = (pe_v[c, pl.ds(g * 16, 16)] & 1) << 6
            nbs = [(plsc.load_gather(ne_v, [jnp.full((16,), 4 * c, jnp.int32)
                                            + (64 * g + 4 * iota + n) // 128,
                                            (64 * g + 4 * iota + n) % 128])
                    & 1) << 6
                   for n in range(NEG_PER_USER)]
            u0 = plsc.load_gather(U, [urow, ub])
            acc_p = u0 + plsc.load_gather(P, [urow, pb + 63])
            accs = [u0 + plsc.load_gather(N, [nr, nb + 63])
                    for nr, nb in zip(nrows, nbs)]
            m = (iota * 5) % 63
            for k in range(EMB - 1):
                ucol = plsc.load_gather(U, [urow, ub + m + 1])
                acc_p = acc_p + ucol * plsc.load_gather(P, [urow, pb + m])
                for n in range(NEG_PER_USER):
                    accs[n] = accs[n] + ucol * plsc.load_gather(
                        N, [nrows[n], nbs[n] + m])
                if k < EMB - 2:
                    m = m + 1
                    m = jnp.where(m == 63, 0, m)
            pos_v[pl.ds(g * 16, 16)] = 1.0 / (1.0 + jnp.exp(-acc_p))
            for n in range(NEG_PER_USER):
                plsc.store_scatter(neg_v, [4 * iota + g * 64 + n],
                                   1.0 / (1.0 + jnp.exp(-accs[n])))
            return 0

        lax.fori_loop(0, SUB // 16, gbody, 0)

        pltpu.sync_copy(pos_v,
                        pos_out.at[pl.ds(w * USERS_PER_WORKER + c * SUB, SUB)])
        pltpu.sync_copy(neg_v,
                        neg_out.at[pl.ds((w * NSUB + c) * (SUB * 4), SUB * 4)])


_k2 = functools.partial(
    pl.kernel,
    mesh=plsc.VectorSubcoreMesh(core_axis_name="c", subcore_axis_name="s"),
    compiler_params=pltpu.CompilerParams(needs_layout_passes=False),
    out_type=(
        jax.ShapeDtypeStruct((B_USERS,), jnp.float32),
        jax.ShapeDtypeStruct((N_NEG,), jnp.float32),
    ),
    scratch_types=[
        pltpu.VMEM((NSUB, IDX_LANES), jnp.int32),
        pltpu.VMEM((NSUB, IDX_LANES), jnp.int32),
        pltpu.VMEM((NSUB * 4, IDX_LANES), jnp.int32),
        pltpu.VMEM((NSUB, IDX_LANES), jnp.int32),
        pltpu.VMEM((NSUB, IDX_LANES), jnp.int32),
        pltpu.VMEM((NSUB * 4, IDX_LANES), jnp.int32),
        pltpu.VMEM((SUB, ROWW), jnp.float32),
        pltpu.VMEM((SUB, ROWW), jnp.float32),
        pltpu.VMEM((SUB * 4, ROWW), jnp.float32),
        pltpu.VMEM((SUB,), jnp.float32),
        pltpu.VMEM((SUB * 4,), jnp.float32),
        pltpu.SemaphoreType.DMA,
    ],
)(_k2_body)


@jax.jit
def kernel(batch_pos, neg_item_index, user_embedding, item_embedding):
    user_num = user_embedding.shape[0]
    uidx = batch_pos[:, 0].astype(jnp.int32)
    pidx = (batch_pos[:, 1] - user_num).astype(jnp.int32)
    nidx = (neg_item_index - user_num).astype(jnp.int32)
    u2, i2 = _k1(user_embedding.T, item_embedding.T)
    r = lambda a: a.reshape(-1, IDX_LANES)
    pos, neg = _k2(r(uidx), r(pidx), r(nidx),
                   r(uidx >> 1), r(pidx >> 1), r(nidx >> 1),
                   u2, i2)
    return pos.reshape(-1, 1), neg.reshape(-1, 1)


# speedup vs baseline: 2.9194x; 1.0512x over previous
import functools

import jax
import jax.numpy as jnp
from jax import lax
from jax.experimental import pallas as pl
from jax.experimental.pallas import tpu as pltpu
from jax.experimental.pallas import tpu_sc as plsc

EMB = 64
NROWS = 1000000
B_USERS = 16384
N_NEG = 65536
NEG_PER_USER = N_NEG // B_USERS
NUM_WORKERS = 32
USERS_PER_WORKER = B_USERS // NUM_WORKERS
SUB = 128
NSUB = USERS_PER_WORKER // SUB
IDX_LANES = 128
ROWW = 2 * EMB
NBLK = (NROWS + 127) // 128
BLK_PER_SUBCORE = (NBLK + 15) // 16


def _relayout_table(tabT, out, Vblk, Oblk, s, semi, semo):
    iota = lax.broadcasted_iota(jnp.int32, (16,), 0)
    kvecs = [i * 16 + iota for i in range(EMB // 16)]
    lo = s * BLK_PER_SUBCORE
    hi = jnp.minimum(NBLK, lo + BLK_PER_SUBCORE)

    def colof(j):
        return pl.multiple_of(j * 128, 128)

    def in_cp(j, slot):
        return pltpu.make_async_copy(
            tabT.at[:, pl.ds(colof(j), 128)], Vblk.at[slot], semi.at[slot])

    def out_cp(j, slot):
        return pltpu.make_async_copy(
            Oblk.at[slot],
            out.at[pl.ds(pl.multiple_of(colof(j) // 2, 64), 64)],
            semo.at[slot])

    def step(j, slot, first):
        in_cp(j, slot).wait()
        if not first:
            out_cp(j, slot).wait()

        Vs, Os = Vblk.at[slot], Oblk.at[slot]

        def dbody(d, m):
            msh = m >> 1
            mpar = (m & 1) << 6
            for t in range(8):
                e0 = t * 16
                ecol = e0 + m
                orow = (e0 >> 1) + msh
                for i in range(EMB // 16):
                    v = plsc.load_gather(Vs, [kvecs[i], ecol])
                    plsc.store_scatter(Os, [orow, mpar + kvecs[i]], v)
            return (m + 1) & 15
        lax.fori_loop(0, 16, dbody, iota)
        out_cp(j, slot).start()

        @pl.when(j + 2 < hi)
        def _():
            in_cp(j + 2, slot).start()

    in_cp(lo, 0).start()
    in_cp(lo + 1, 1).start()
    step(lo, 0, True)
    step(lo + 1, 1, True)

    def jbody(jj, _):
        j = lo + 2 * jj

        @pl.when(j < hi)
        def _():
            step(j, 0, False)

        @pl.when(j + 1 < hi)
        def _():
            step(j + 1, 1, False)

        return 0

    lax.fori_loop(1, (BLK_PER_SUBCORE + 1) // 2, jbody, 0)
    out_cp(hi - 2, 0).wait()
    out_cp(hi - 1, 1).wait()


def _k1_body(uT_hbm, iT_hbm, u_out, i_out, Vblk, Oblk, semi, semo):
    c = lax.axis_index("c")
    s = lax.axis_index("s")

    @pl.when(c == 0)
    def _():
        _relayout_table(uT_hbm, u_out, Vblk, Oblk, s, semi, semo)

    @pl.when(c == 1)
    def _():
        _relayout_table(iT_hbm, i_out, Vblk, Oblk, s, semi, semo)


_k1 = functools.partial(
    pl.kernel,
    mesh=plsc.VectorSubcoreMesh(core_axis_name="c", subcore_axis_name="s"),
    compiler_params=pltpu.CompilerParams(needs_layout_passes=False),
    out_type=(
        jax.ShapeDtypeStruct((NBLK * 64, ROWW), jnp.float32),
        jax.ShapeDtypeStruct((NBLK * 64, ROWW), jnp.float32),
    ),
    scratch_types=[
        pltpu.VMEM((2, EMB, 128), jnp.float32),
        pltpu.VMEM((2, 64, ROWW), jnp.float32),
        pltpu.SemaphoreType.DMA((2,)),
        pltpu.SemaphoreType.DMA((2,)),
    ],
)(_k1_body)


def _k2_body(ue_hbm, pe_hbm, ne_hbm, ur_hbm, pr_hbm, nr_hbm,
             uemb_hbm, iemb_hbm, pos_out, neg_out,
             ue_v, pe_v, ne_v, ur_v, pr_v, nr_v, U, P, N, pos_v, neg_v, sem):
    w = lax.axis_index("s") * 2 + lax.axis_index("c")
    iota = lax.broadcasted_iota(jnp.int32, (16,), 0)

    pltpu.sync_copy(ue_hbm.at[pl.ds(w * NSUB, NSUB)], ue_v)
    pltpu.sync_copy(pe_hbm.at[pl.ds(w * NSUB, NSUB)], pe_v)
    pltpu.sync_copy(ne_hbm.at[pl.ds(w * (NSUB * 4), NSUB * 4)], ne_v)
    pltpu.sync_copy(ur_hbm.at[pl.ds(w * NSUB, NSUB)], ur_v)
    pltpu.sync_copy(pr_hbm.at[pl.ds(w * NSUB, NSUB)], pr_v)
    pltpu.sync_copy(nr_hbm.at[pl.ds(w * (NSUB * 4), NSUB * 4)], nr_v)

    for c in range(NSUB):
        cps = [
            pltpu.async_copy(uemb_hbm.at[ur_v.at[c]], U, sem),
            pltpu.async_copy(iemb_hbm.at[pr_v.at[c]], P, sem),
        ]
        for j in range(4):
            cps.append(pltpu.async_copy(
                iemb_hbm.at[nr_v.at[4 * c + j]],
                N.at[pl.ds(IDX_LANES * j, IDX_LANES)], sem))
        for cp in cps:
            cp.wait()

        def gbody(g, _):
            urow = iota + g * 16
            nrows = [4 * iota + g * 64 + n for n in range(NEG_PER_USER)]
            ub = (ue_v[c, pl.ds(g * 16, 16)] & 1) << 6
            pb = (pe_v[c, pl.ds(g * 16, 16)] & 1) << 6
            nbs = [(plsc.load_gather(ne_v, [jnp.full((16,), 4 * c, jnp.int32)
                                            + (64 * g + 4 * iota + n) // 128,
                                            (64 * g + 4 * iota + n) % 128])
                    & 1) << 6
                   for n in range(NEG_PER_USER)]
            u0 = plsc.load_gather(U, [urow, ub])
            acc_p = u0 + plsc.load_gather(P, [urow, pb + 63])
            accs = [u0 + plsc.load_gather(N, [nr, nb + 63])
                    for nr, nb in zip(nrows, nbs)]
            m = (iota * 5) % 63
            for k in range(EMB - 1):
                ucol = plsc.load_gather(U, [urow, ub + m + 1])
                acc_p = acc_p + ucol * plsc.load_gather(P, [urow, pb + m])
                for n in range(NEG_PER_USER):
                    accs[n] = accs[n] + ucol * plsc.load_gather(
                        N, [nrows[n], nbs[n] + m])
                if k < EMB - 2:
                    m = m + 1
                    m = jnp.where(m == 63, 0, m)
            pos_v[pl.ds(g * 16, 16)] = 1.0 / (1.0 + jnp.exp(-acc_p))
            for n in range(NEG_PER_USER):
                plsc.store_scatter(neg_v, [4 * iota + g * 64 + n],
                                   1.0 / (1.0 + jnp.exp(-accs[n])))
            return 0

        lax.fori_loop(0, SUB // 16, gbody, 0)

        pltpu.sync_copy(pos_v,
                        pos_out.at[pl.ds(w * USERS_PER_WORKER + c * SUB, SUB)])
        pltpu.sync_copy(neg_v,
                        neg_out.at[pl.ds((w * NSUB + c) * (SUB * 4), SUB * 4)])


_k2 = functools.partial(
    pl.kernel,
    mesh=plsc.VectorSubcoreMesh(core_axis_name="c", subcore_axis_name="s"),
    compiler_params=pltpu.CompilerParams(needs_layout_passes=False),
    out_type=(
        jax.ShapeDtypeStruct((B_USERS,), jnp.float32),
        jax.ShapeDtypeStruct((N_NEG,), jnp.float32),
    ),
    scratch_types=[
        pltpu.VMEM((NSUB, IDX_LANES), jnp.int32),
        pltpu.VMEM((NSUB, IDX_LANES), jnp.int32),
        pltpu.VMEM((NSUB * 4, IDX_LANES), jnp.int32),
        pltpu.VMEM((NSUB, IDX_LANES), jnp.int32),
        pltpu.VMEM((NSUB, IDX_LANES), jnp.int32),
        pltpu.VMEM((NSUB * 4, IDX_LANES), jnp.int32),
        pltpu.VMEM((SUB, ROWW), jnp.float32),
        pltpu.VMEM((SUB, ROWW), jnp.float32),
        pltpu.VMEM((SUB * 4, ROWW), jnp.float32),
        pltpu.VMEM((SUB,), jnp.float32),
        pltpu.VMEM((SUB * 4,), jnp.float32),
        pltpu.SemaphoreType.DMA,
    ],
)(_k2_body)


@jax.jit
def kernel(batch_pos, neg_item_index, user_embedding, item_embedding):
    user_num = user_embedding.shape[0]
    uidx = batch_pos[:, 0].astype(jnp.int32)
    pidx = (batch_pos[:, 1] - user_num).astype(jnp.int32)
    nidx = (neg_item_index - user_num).astype(jnp.int32)
    u2, i2 = _k1(user_embedding.T, item_embedding.T)
    r = lambda a: a.reshape(-1, IDX_LANES)
    pos, neg = _k2(r(uidx), r(pidx), r(nidx),
                   r(uidx >> 1), r(pidx >> 1), r(nidx >> 1),
                   u2, i2)
    return pos.reshape(-1, 1), neg.reshape(-1, 1)
